# Initial kernel scaffold; baseline (speedup 1.0000x reference)
import jax
import jax.numpy as jnp
from jax.experimental import pallas as pl


def kernel(f_atoms, f_bonds, a2b, b2a, b2revb, atom_mol_ids, ano_f_atoms, ano_f_bonds, ano_a2b, ano_b2a, ano_b2revb, ano_atom_mol_ids, W_i, W_h, W_o, b_o):
    raise NotImplementedError("write your pallas kernel here")


# speedup vs baseline: 1.2579x; 1.2579x over previous
import functools

import jax
import jax.numpy as jnp
from jax import lax
from jax.experimental import pallas as pl
from jax.experimental.pallas import tpu as pltpu
from jax.experimental.pallas import tpu_sc as plsc

_NC = 2
_NS = 16
_NW = _NC * _NS
_LANES = 16
_H = 128
_CHUNK = 128


def _sc_mesh():
    return plsc.VectorSubcoreMesh(
        core_axis_name="c", subcore_axis_name="s", num_cores=_NC,
        num_subcores=_NS)


def _tree_sum(vals):
    while len(vals) > 1:
        nxt = [vals[i] + vals[i + 1] for i in range(0, len(vals) - 1, 2)]
        if len(vals) % 2:
            nxt.append(vals[-1])
        vals = nxt
    return vals[0]


def _nbsum(table, idx_flat, n_nb, interpret=False):
    n_idx = idx_flat.shape[0]
    assert n_idx % _CHUNK == 0 and _CHUNK % n_nb == 0
    n_chunks = n_idx // _CHUNK
    apc = _CHUNK // n_nb
    n_out = n_idx // n_nb
    n_iters = (n_chunks + _NW - 1) // _NW

    @functools.partial(
        pl.kernel,
        out_type=jax.ShapeDtypeStruct((n_out, _H), jnp.float32),
        mesh=_sc_mesh(),
        scratch_types=[
            pltpu.VMEM((_CHUNK,), jnp.int32),
            pltpu.VMEM((_CHUNK, _H), jnp.float32),
            pltpu.VMEM((apc, _H), jnp.float32),
            pltpu.SemaphoreType.DMA,
        ],
        interpret=interpret,
    )
    def k(tbl_hbm, idx_hbm, out_hbm, idx_v, rows_v, acc_v, sem):
        wid = lax.axis_index("s") * _NC + lax.axis_index("c")

        def body(i, carry):
            c = wid + i * _NW

            @pl.when(c < n_chunks)
            def _():
                pltpu.sync_copy(idx_hbm.at[pl.ds(c * _CHUNK, _CHUNK)], idx_v)
                pltpu.async_copy(tbl_hbm.at[idx_v], rows_v, sem).wait()
                for j in range(_H // _LANES):
                    sl = pl.ds(j * _LANES, _LANES)
                    for r in range(apc):
                        acc_v[r, sl] = _tree_sum(
                            [rows_v[r * n_nb + t, sl] for t in range(n_nb)])
                pltpu.sync_copy(acc_v, out_hbm.at[pl.ds(c * apc, apc)])

            return carry

        lax.fori_loop(0, n_iters, body, 0)

    return k(table, idx_flat)


def _pairmsg(inp, p, a, b2a, b2revb, interpret=False):
    n_bonds = inp.shape[0]
    assert n_bonds % _CHUNK == 0
    n_chunks = n_bonds // _CHUNK
    n_iters = (n_chunks + _NW - 1) // _NW

    @functools.partial(
        pl.kernel,
        out_type=jax.ShapeDtypeStruct((n_bonds, _H), jnp.float32),
        mesh=_sc_mesh(),
        scratch_types=[
            pltpu.VMEM((_CHUNK,), jnp.int32),
            pltpu.VMEM((_CHUNK,), jnp.int32),
            pltpu.VMEM((_CHUNK, _H), jnp.float32),
            pltpu.VMEM((_CHUNK, _H), jnp.float32),
            pltpu.VMEM((_CHUNK, _H), jnp.float32),
            pltpu.VMEM((_CHUNK, _H), jnp.float32),
            pltpu.SemaphoreType.DMA,
        ],
        interpret=interpret,
    )
    def k(inp_hbm, p_hbm, a_hbm, b2a_hbm, b2revb_hbm, out_hbm,
          idxa_v, idxr_v, rowsa_v, rowsr_v, inp_v, out_v, sem):
        wid = lax.axis_index("s") * _NC + lax.axis_index("c")

        def body(i, carry):
            c = wid + i * _NW

            @pl.when(c < n_chunks)
            def _():
                base = pl.ds(c * _CHUNK, _CHUNK)
                pltpu.sync_copy(b2a_hbm.at[base], idxa_v)
                pltpu.sync_copy(b2revb_hbm.at[base], idxr_v)
                ca = pltpu.async_copy(a_hbm.at[idxa_v], rowsa_v, sem)
                cr = pltpu.async_copy(p_hbm.at[idxr_v], rowsr_v, sem)
                ci = pltpu.async_copy(inp_hbm.at[base], inp_v, sem)
                ca.wait()
                cr.wait()
                ci.wait()

                def rbody(r, rc):
                    for j in range(_H // _LANES):
                        sl = pl.ds(j * _LANES, _LANES)
                        v = inp_v[r, sl] + rowsa_v[r, sl] - rowsr_v[r, sl]
                        out_v[r, sl] = jnp.maximum(v, 0.0)
                    return rc

                lax.fori_loop(0, _CHUNK, rbody, 0)
                pltpu.sync_copy(out_v, out_hbm.at[base])

            return carry

        lax.fori_loop(0, n_iters, body, 0)

    return k(inp, p, a, b2a, b2revb)


def _mm_in(f_bonds, w_i, w_h, interpret=False):
    n, kdim = f_bonds.shape
    br = 2560
    assert n % br == 0

    def body(x_ref, wi_ref, wh_ref, inp_ref, p0_ref):
        i = jnp.dot(x_ref[...], wi_ref[...],
                    preferred_element_type=jnp.float32)
        inp_ref[...] = i
        p0_ref[...] = jnp.dot(jnp.maximum(i, 0.0), wh_ref[...],
                              preferred_element_type=jnp.float32)

    return pl.pallas_call(
        body,
        grid=(n // br,),
        in_specs=[
            pl.BlockSpec((br, kdim), lambda i: (i, 0)),
            pl.BlockSpec((kdim, _H), lambda i: (0, 0)),
            pl.BlockSpec((_H, _H), lambda i: (0, 0)),
        ],
        out_specs=[
            pl.BlockSpec((br, _H), lambda i: (i, 0)),
            pl.BlockSpec((br, _H), lambda i: (i, 0)),
        ],
        out_shape=[
            jax.ShapeDtypeStruct((n, _H), jnp.float32),
            jax.ShapeDtypeStruct((n, _H), jnp.float32),
        ],
        interpret=interpret,
    )(f_bonds, w_i, w_h)


def _mm(x, w, interpret=False):
    n, kdim = x.shape
    br = 2560
    assert n % br == 0

    def body(x_ref, w_ref, o_ref):
        o_ref[...] = jnp.dot(x_ref[...], w_ref[...],
                             preferred_element_type=jnp.float32)

    return pl.pallas_call(
        body,
        grid=(n // br,),
        in_specs=[
            pl.BlockSpec((br, kdim), lambda i: (i, 0)),
            pl.BlockSpec((kdim, _H), lambda i: (0, 0)),
        ],
        out_specs=pl.BlockSpec((br, _H), lambda i: (i, 0)),
        out_shape=jax.ShapeDtypeStruct((n, _H), jnp.float32),
        interpret=interpret,
    )(x, w)


def _readout(f_atoms, a_msg, mol_ids_f32, w_o1, w_o2, b_o, n_mols,
             interpret=False):
    n_atoms = f_atoms.shape[0]
    br = 2000
    assert n_atoms % br == 0
    steps = n_atoms // br

    def body(fa_ref, am_ref, mid_ref, wo1_ref, wo2_ref, bo_ref, out_ref,
             sum_v, cnt_v):
        step = pl.program_id(0)

        @pl.when(step == 0)
        def _():
            sum_v[...] = jnp.zeros_like(sum_v)
            cnt_v[...] = jnp.zeros_like(cnt_v)

        h = jnp.maximum(
            jnp.dot(fa_ref[...], wo1_ref[...],
                    preferred_element_type=jnp.float32)
            + jnp.dot(am_ref[...], wo2_ref[...],
                      preferred_element_type=jnp.float32)
            + bo_ref[...], 0.0)
        iota = lax.broadcasted_iota(
            jnp.int32, (1, n_mols), 1).astype(jnp.float32)
        oh = (mid_ref[...] == iota).astype(jnp.float32)
        sum_v[...] += lax.dot_general(
            oh, h, (((0,), (0,)), ((), ())),
            preferred_element_type=jnp.float32)
        cnt_v[...] += lax.dot_general(
            oh, jnp.ones((br, 1), jnp.float32), (((0,), (0,)), ((), ())),
            preferred_element_type=jnp.float32)

        @pl.when(step == steps - 1)
        def _():
            out_ref[...] = sum_v[...] / jnp.maximum(cnt_v[...], 1.0)

    return pl.pallas_call(
        body,
        grid=(steps,),
        in_specs=[
            pl.BlockSpec((br, _H), lambda i: (i, 0)),
            pl.BlockSpec((br, _H), lambda i: (i, 0)),
            pl.BlockSpec((br, 1), lambda i: (i, 0)),
            pl.BlockSpec((_H, _H), lambda i: (0, 0)),
            pl.BlockSpec((_H, _H), lambda i: (0, 0)),
            pl.BlockSpec((1, _H), lambda i: (0, 0)),
        ],
        out_specs=pl.BlockSpec((n_mols, _H), lambda i: (0, 0)),
        out_shape=jax.ShapeDtypeStruct((n_mols, _H), jnp.float32),
        scratch_shapes=[
            pltpu.VMEM((n_mols, _H), jnp.float32),
            pltpu.VMEM((n_mols, 1), jnp.float32),
        ],
        interpret=interpret,
    )(f_atoms, a_msg, mol_ids_f32, w_o1, w_o2, b_o)


def _encode(f_atoms, f_bonds, a2b, b2a, b2revb, mol_ids, w_i, w_h, w_o, b_o,
            n_mols=256):
    n_nb = a2b.shape[1]
    a2b_flat = a2b.reshape(-1).astype(jnp.int32)
    b2a = b2a.astype(jnp.int32)
    b2revb = b2revb.astype(jnp.int32)

    inp, p0 = _mm_in(f_bonds, w_i, w_h)
    a0 = _nbsum(p0, a2b_flat, n_nb)
    msg1 = _pairmsg(inp, p0, a0, b2a, b2revb)
    p1 = _mm(msg1, w_h)
    a1 = _nbsum(p1, a2b_flat, n_nb)
    msg2 = _pairmsg(inp, p1, a1, b2a, b2revb)
    a2 = _nbsum(msg2, a2b_flat, n_nb)

    mid = mol_ids.astype(jnp.float32).reshape(-1, 1)
    w_o1 = w_o[:f_atoms.shape[1], :]
    w_o2 = w_o[f_atoms.shape[1]:, :]
    return _readout(f_atoms, a2, mid, w_o1, w_o2, b_o.reshape(1, -1), n_mols)


def kernel(f_atoms, f_bonds, a2b, b2a, b2revb, atom_mol_ids,
           ano_f_atoms, ano_f_bonds, ano_a2b, ano_b2a, ano_b2revb,
           ano_atom_mol_ids, W_i, W_h, W_o, b_o):
    mol_vecs = _encode(f_atoms, f_bonds, a2b, b2a, b2revb, atom_mol_ids,
                       W_i, W_h, W_o, b_o)
    ano_mol_vecs = _encode(ano_f_atoms, ano_f_bonds, ano_a2b, ano_b2a,
                           ano_b2revb, ano_atom_mol_ids, W_i, W_h, W_o, b_o)
    return (mol_vecs, ano_mol_vecs)


# speedup vs baseline: 2.3227x; 1.8465x over previous
import functools

import jax
import jax.numpy as jnp
from jax import lax
from jax.experimental import pallas as pl
from jax.experimental.pallas import tpu as pltpu
from jax.experimental.pallas import tpu_sc as plsc

_NC = 2
_NS = 16
_NW = _NC * _NS
_LANES = 16
_H = 128
_CHUNK = 128


def _sc_mesh():
    return plsc.VectorSubcoreMesh(
        core_axis_name="c", subcore_axis_name="s", num_cores=_NC,
        num_subcores=_NS)


def _tree_sum(vals):
    while len(vals) > 1:
        nxt = [vals[i] + vals[i + 1] for i in range(0, len(vals) - 1, 2)]
        if len(vals) % 2:
            nxt.append(vals[-1])
        vals = nxt
    return vals[0]


def _split_chunks(wid, n_chunks):
    base = n_chunks // _NW
    rem = n_chunks % _NW
    start = base * wid + jnp.minimum(wid, rem)
    n_ch = base + jnp.where(wid < rem, 1, 0)
    return start, n_ch, base + (1 if rem else 0)


def _stage_idx(idx_hbm, idx_v, start, n_ch, base, cpb):
    src0 = pl.multiple_of(start * cpb, 128)
    pltpu.sync_copy(idx_hbm.at[pl.ds(src0, base * cpb)],
                    idx_v.at[pl.ds(0, base * cpb)])

    @pl.when(n_ch > base)
    def _():
        src1 = pl.multiple_of((start + base) * cpb, 128)
        pltpu.sync_copy(idx_hbm.at[pl.ds(src1, cpb)],
                        idx_v.at[pl.ds(base * cpb, cpb)])


def _nbsum(table, idx_flat, n_nb, interpret=False):
    cpb = 2 * _CHUNK
    n_chunks = idx_flat.shape[0] // cpb
    apc = cpb // n_nb
    n_out = n_chunks * apc
    base = n_chunks // _NW

    @functools.partial(
        pl.kernel,
        out_type=jax.ShapeDtypeStruct((n_out, _H), jnp.float32),
        mesh=_sc_mesh(),
        scratch_types=[
            pltpu.VMEM(((base + 1) * cpb,), jnp.int32),
            pltpu.VMEM((2, cpb, _H), jnp.float32),
            pltpu.VMEM((2, apc, _H), jnp.float32),
            pltpu.SemaphoreType.DMA,
            pltpu.SemaphoreType.DMA,
            pltpu.SemaphoreType.DMA,
            pltpu.SemaphoreType.DMA,
        ],
        interpret=interpret,
    )
    def k(tbl_hbm, idx_hbm, out_hbm, idx_v, rows_v, acc_v,
          gsem0, gsem1, wsem0, wsem1):
        wid = lax.axis_index("s") * _NC + lax.axis_index("c")
        start, n_ch, max_ch = _split_chunks(wid, n_chunks)
        gsem = (gsem0, gsem1)
        wsem = (wsem0, wsem1)
        _stage_idx(idx_hbm, idx_v, start, n_ch, base, cpb)

        def issue_gather(kc, slot):
            @pl.when(kc < n_ch)
            def _():
                for g in range(cpb // _CHUNK):
                    off = pl.multiple_of(kc * cpb + g * _CHUNK, 128)
                    pltpu.async_copy(
                        tbl_hbm.at[idx_v.at[pl.ds(off, _CHUNK)]],
                        rows_v.at[slot, pl.ds(g * _CHUNK, _CHUNK)],
                        gsem[slot])

        def drain_gather(slot):
            for g in range(cpb // _CHUNK):
                pltpu.make_async_copy(
                    tbl_hbm.at[pl.ds(0, _CHUNK)],
                    rows_v.at[slot, pl.ds(g * _CHUNK, _CHUNK)],
                    gsem[slot]).wait()

        def drain_write(slot):
            pltpu.make_async_copy(acc_v.at[slot],
                                  out_hbm.at[pl.ds(0, apc)],
                                  wsem[slot]).wait()

        issue_gather(0, 0)

        def body(i, carry):
            for s in (0, 1):
                kc = 2 * i + s

                @pl.when(kc < n_ch)
                def _():
                    issue_gather(kc + 1, (s + 1) % 2)
                    drain_gather(s)

                    @pl.when(kc >= 2)
                    def _():
                        drain_write(s)

                    def rbody(r, rc):
                        for j in range(_H // _LANES):
                            sl = pl.ds(j * _LANES, _LANES)
                            acc_v[s, r, sl] = _tree_sum(
                                [rows_v[s, r * n_nb + t, sl]
                                 for t in range(n_nb)])
                        return rc

                    lax.fori_loop(0, apc, rbody, 0)
                    orow = pl.multiple_of((start + kc) * apc, 8)
                    pltpu.async_copy(acc_v.at[s],
                                     out_hbm.at[pl.ds(orow, apc)], wsem[s])

            return carry

        lax.fori_loop(0, (max_ch + 1) // 2, body, 0)
        drain_write(0)
        drain_write(1)

    return k(table, idx_flat)


def _pairmsg(inp, p, a, b2a, b2revb, interpret=False):
    n_bonds = inp.shape[0]
    n_chunks = n_bonds // _CHUNK
    base = n_chunks // _NW

    @functools.partial(
        pl.kernel,
        out_type=jax.ShapeDtypeStruct((n_bonds, _H), jnp.float32),
        mesh=_sc_mesh(),
        scratch_types=[
            pltpu.VMEM(((base + 1) * _CHUNK,), jnp.int32),
            pltpu.VMEM(((base + 1) * _CHUNK,), jnp.int32),
            pltpu.VMEM((2, _CHUNK, _H), jnp.float32),
            pltpu.VMEM((2, _CHUNK, _H), jnp.float32),
            pltpu.VMEM((2, _CHUNK, _H), jnp.float32),
            pltpu.SemaphoreType.DMA,
            pltpu.SemaphoreType.DMA,
            pltpu.SemaphoreType.DMA,
            pltpu.SemaphoreType.DMA,
        ],
        interpret=interpret,
    )
    def k(inp_hbm, p_hbm, a_hbm, b2a_hbm, b2revb_hbm, out_hbm,
          idxa_v, idxr_v, rowsa_v, rowsr_v, inp_v,
          gsem0, gsem1, wsem0, wsem1):
        wid = lax.axis_index("s") * _NC + lax.axis_index("c")
        start, n_ch, max_ch = _split_chunks(wid, n_chunks)
        gsem = (gsem0, gsem1)
        wsem = (wsem0, wsem1)
        _stage_idx(b2a_hbm, idxa_v, start, n_ch, base, _CHUNK)
        _stage_idx(b2revb_hbm, idxr_v, start, n_ch, base, _CHUNK)

        def issue_ar(kc, slot):
            @pl.when(kc < n_ch)
            def _():
                off = pl.multiple_of(kc * _CHUNK, 128)
                pltpu.async_copy(a_hbm.at[idxa_v.at[pl.ds(off, _CHUNK)]],
                                 rowsa_v.at[slot], gsem[slot])
                pltpu.async_copy(p_hbm.at[idxr_v.at[pl.ds(off, _CHUNK)]],
                                 rowsr_v.at[slot], gsem[slot])

        def issue_inp(kc, slot):
            @pl.when(kc < n_ch)
            def _():
                row = pl.multiple_of((start + kc) * _CHUNK, 128)
                pltpu.async_copy(inp_hbm.at[pl.ds(row, _CHUNK)],
                                 inp_v.at[slot], gsem[slot])

        def drain_gathers(slot):
            dummy = pl.ds(0, _CHUNK)
            pltpu.make_async_copy(p_hbm.at[dummy], rowsa_v.at[slot],
                                  gsem[slot]).wait()
            pltpu.make_async_copy(p_hbm.at[dummy], rowsr_v.at[slot],
                                  gsem[slot]).wait()
            pltpu.make_async_copy(p_hbm.at[dummy], inp_v.at[slot],
                                  gsem[slot]).wait()

        def drain_write(slot):
            pltpu.make_async_copy(inp_v.at[slot],
                                  out_hbm.at[pl.ds(0, _CHUNK)],
                                  wsem[slot]).wait()

        issue_ar(0, 0)
        issue_inp(0, 0)

        def body(i, carry):
            for s in (0, 1):
                kc = 2 * i + s
                s2 = (s + 1) % 2

                @pl.when(kc < n_ch)
                def _():
                    issue_ar(kc + 1, s2)
                    drain_gathers(s)

                    @pl.when(jnp.logical_and(kc >= 1, kc + 1 < n_ch))
                    def _():
                        drain_write(s2)

                    issue_inp(kc + 1, s2)

                    def rbody(r, rc):
                        for j in range(_H // _LANES):
                            sl = pl.ds(j * _LANES, _LANES)
                            v = (inp_v[s, r, sl] + rowsa_v[s, r, sl]
                                 - rowsr_v[s, r, sl])
                            inp_v[s, r, sl] = jnp.maximum(v, 0.0)
                        return rc

                    lax.fori_loop(0, _CHUNK, rbody, 0)
                    row = pl.multiple_of((start + kc) * _CHUNK, 128)
                    pltpu.async_copy(inp_v.at[s],
                                     out_hbm.at[pl.ds(row, _CHUNK)], wsem[s])

            return carry

        lax.fori_loop(0, (max_ch + 1) // 2, body, 0)
        drain_write(0)
        drain_write(1)

    return k(inp, p, a, b2a, b2revb)


def _mm_in(f_bonds, w_i, w_h, interpret=False):
    n, kdim = f_bonds.shape
    br = 2560
    assert n % br == 0

    def body(x_ref, wi_ref, wh_ref, inp_ref, p0_ref):
        i = jnp.dot(x_ref[...], wi_ref[...],
                    preferred_element_type=jnp.float32)
        inp_ref[...] = i
        p0_ref[...] = jnp.dot(jnp.maximum(i, 0.0), wh_ref[...],
                              preferred_element_type=jnp.float32)

    return pl.pallas_call(
        body,
        grid=(n // br,),
        in_specs=[
            pl.BlockSpec((br, kdim), lambda i: (i, 0)),
            pl.BlockSpec((kdim, _H), lambda i: (0, 0)),
            pl.BlockSpec((_H, _H), lambda i: (0, 0)),
        ],
        out_specs=[
            pl.BlockSpec((br, _H), lambda i: (i, 0)),
            pl.BlockSpec((br, _H), lambda i: (i, 0)),
        ],
        out_shape=[
            jax.ShapeDtypeStruct((n, _H), jnp.float32),
            jax.ShapeDtypeStruct((n, _H), jnp.float32),
        ],
        interpret=interpret,
    )(f_bonds, w_i, w_h)


def _mm(x, w, interpret=False):
    n, kdim = x.shape
    br = 2560
    assert n % br == 0

    def body(x_ref, w_ref, o_ref):
        o_ref[...] = jnp.dot(x_ref[...], w_ref[...],
                             preferred_element_type=jnp.float32)

    return pl.pallas_call(
        body,
        grid=(n // br,),
        in_specs=[
            pl.BlockSpec((br, kdim), lambda i: (i, 0)),
            pl.BlockSpec((kdim, _H), lambda i: (0, 0)),
        ],
        out_specs=pl.BlockSpec((br, _H), lambda i: (i, 0)),
        out_shape=jax.ShapeDtypeStruct((n, _H), jnp.float32),
        interpret=interpret,
    )(x, w)


def _readout(f_atoms, a_msg, mol_ids_f32, w_o1, w_o2, b_o, n_mols,
             interpret=False):
    n_atoms = f_atoms.shape[0]
    br = 2000
    assert n_atoms % br == 0
    steps = n_atoms // br

    def body(fa_ref, am_ref, mid_ref, wo1_ref, wo2_ref, bo_ref, out_ref,
             sum_v, cnt_v):
        step = pl.program_id(0)

        @pl.when(step == 0)
        def _():
            sum_v[...] = jnp.zeros_like(sum_v)
            cnt_v[...] = jnp.zeros_like(cnt_v)

        h = jnp.maximum(
            jnp.dot(fa_ref[...], wo1_ref[...],
                    preferred_element_type=jnp.float32)
            + jnp.dot(am_ref[...], wo2_ref[...],
                      preferred_element_type=jnp.float32)
            + bo_ref[...], 0.0)
        iota = lax.broadcasted_iota(
            jnp.int32, (1, n_mols), 1).astype(jnp.float32)
        oh = (mid_ref[...] == iota).astype(jnp.float32)
        sum_v[...] += lax.dot_general(
            oh, h, (((0,), (0,)), ((), ())),
            preferred_element_type=jnp.float32)
        cnt_v[...] += lax.dot_general(
            oh, jnp.ones((br, 1), jnp.float32), (((0,), (0,)), ((), ())),
            preferred_element_type=jnp.float32)

        @pl.when(step == steps - 1)
        def _():
            out_ref[...] = sum_v[...] / jnp.maximum(cnt_v[...], 1.0)

    return pl.pallas_call(
        body,
        grid=(steps,),
        in_specs=[
            pl.BlockSpec((br, _H), lambda i: (i, 0)),
            pl.BlockSpec((br, _H), lambda i: (i, 0)),
            pl.BlockSpec((br, 1), lambda i: (i, 0)),
            pl.BlockSpec((_H, _H), lambda i: (0, 0)),
            pl.BlockSpec((_H, _H), lambda i: (0, 0)),
            pl.BlockSpec((1, _H), lambda i: (0, 0)),
        ],
        out_specs=pl.BlockSpec((n_mols, _H), lambda i: (0, 0)),
        out_shape=jax.ShapeDtypeStruct((n_mols, _H), jnp.float32),
        scratch_shapes=[
            pltpu.VMEM((n_mols, _H), jnp.float32),
            pltpu.VMEM((n_mols, 1), jnp.float32),
        ],
        interpret=interpret,
    )(f_atoms, a_msg, mol_ids_f32, w_o1, w_o2, b_o)


def _encode(f_atoms, f_bonds, a2b, b2a, b2revb, mol_ids, w_i, w_h, w_o, b_o,
            n_mols=256):
    n_nb = a2b.shape[1]
    a2b_flat = a2b.reshape(-1).astype(jnp.int32)
    b2a = b2a.astype(jnp.int32)
    b2revb = b2revb.astype(jnp.int32)

    inp, p0 = _mm_in(f_bonds, w_i, w_h)
    a0 = _nbsum(p0, a2b_flat, n_nb)
    msg1 = _pairmsg(inp, p0, a0, b2a, b2revb)
    p1 = _mm(msg1, w_h)
    a1 = _nbsum(p1, a2b_flat, n_nb)
    msg2 = _pairmsg(inp, p1, a1, b2a, b2revb)
    a2 = _nbsum(msg2, a2b_flat, n_nb)

    mid = mol_ids.astype(jnp.float32).reshape(-1, 1)
    w_o1 = w_o[:f_atoms.shape[1], :]
    w_o2 = w_o[f_atoms.shape[1]:, :]
    return _readout(f_atoms, a2, mid, w_o1, w_o2, b_o.reshape(1, -1), n_mols)


def kernel(f_atoms, f_bonds, a2b, b2a, b2revb, atom_mol_ids,
           ano_f_atoms, ano_f_bonds, ano_a2b, ano_b2a, ano_b2revb,
           ano_atom_mol_ids, W_i, W_h, W_o, b_o):
    mol_vecs = _encode(f_atoms, f_bonds, a2b, b2a, b2revb, atom_mol_ids,
                       W_i, W_h, W_o, b_o)
    ano_mol_vecs = _encode(ano_f_atoms, ano_f_bonds, ano_a2b, ano_b2a,
                           ano_b2revb, ano_atom_mol_ids, W_i, W_h, W_o, b_o)
    return (mol_vecs, ano_mol_vecs)


# speedup vs baseline: 2.5819x; 1.1116x over previous
import functools

import jax
import jax.numpy as jnp
from jax import lax
from jax.experimental import pallas as pl
from jax.experimental.pallas import tpu as pltpu
from jax.experimental.pallas import tpu_sc as plsc

_NC = 2
_NS = 16
_NW = _NC * _NS
_LANES = 16
_H = 128
_HP = _H // 2
_CHUNK = 128

_SC_PARAMS = pltpu.CompilerParams(use_tc_tiling_on_sc=False)


def _sc_mesh():
    return plsc.VectorSubcoreMesh(
        core_axis_name="c", subcore_axis_name="s", num_cores=_NC,
        num_subcores=_NS)


def _tree_sum(vals):
    while len(vals) > 1:
        nxt = [vals[i] + vals[i + 1] for i in range(0, len(vals) - 1, 2)]
        if len(vals) % 2:
            nxt.append(vals[-1])
        vals = nxt
    return vals[0]


def _sc_unpack(x):
    lo = lax.bitcast_convert_type(lax.shift_left(x, 16), jnp.float32)
    hi = lax.bitcast_convert_type(
        jnp.bitwise_and(x, jnp.int32(-65536)), jnp.float32)
    return lo, hi


def _sc_pack(lo, hi):
    lob = lax.shift_right_logical(
        lax.bitcast_convert_type(lo, jnp.int32) + 32768, 16)
    hib = jnp.bitwise_and(
        lax.bitcast_convert_type(hi, jnp.int32) + 32768, jnp.int32(-65536))
    return jnp.bitwise_or(lob, hib)


def _tc_pack(x):
    lo = lax.bitcast_convert_type(
        x[:, :_HP].astype(jnp.bfloat16), jnp.uint16).astype(jnp.uint32)
    hi = lax.bitcast_convert_type(
        x[:, _HP:].astype(jnp.bfloat16), jnp.uint16).astype(jnp.uint32)
    return lax.bitcast_convert_type(lo | (hi << 16), jnp.int32)


def _tc_unpack(xi):
    lo = lax.bitcast_convert_type(lax.shift_left(xi, 16), jnp.float32)
    hi = lax.bitcast_convert_type(
        jnp.bitwise_and(xi, jnp.int32(-65536)), jnp.float32)
    return lo, hi


def _split_chunks(wid, n_chunks):
    b = n_chunks // _NW
    rem = n_chunks % _NW
    start = b * wid + jnp.minimum(wid, rem)
    n_ch = b + jnp.where(wid < rem, 1, 0)
    return start, n_ch, b + (1 if rem else 0)


def _stage_idx(idx_hbm, idx_v, start, n_ch, b, cpb):
    src0 = pl.multiple_of(start * cpb, 128)
    pltpu.sync_copy(idx_hbm.at[pl.ds(src0, b * cpb)],
                    idx_v.at[pl.ds(0, b * cpb)])

    @pl.when(n_ch > b)
    def _():
        src1 = pl.multiple_of((start + b) * cpb, 128)
        pltpu.sync_copy(idx_hbm.at[pl.ds(src1, cpb)],
                        idx_v.at[pl.ds(b * cpb, cpb)])


def _nbsum(table64, idxg_flat, n_nb, interpret=False):
    cpb = 2 * _CHUNK
    n_chunks = idxg_flat.shape[0] // cpb
    apc = cpb // n_nb
    n_out = n_chunks * apc
    b = n_chunks // _NW

    @functools.partial(
        pl.kernel,
        out_type=jax.ShapeDtypeStruct((n_out, _HP), jnp.int32),
        mesh=_sc_mesh(),
        compiler_params=_SC_PARAMS,
        scratch_types=[
            pltpu.VMEM(((b + 1) * cpb,), jnp.int32),
            pltpu.VMEM((2, cpb, _HP), jnp.int32),
            pltpu.VMEM((2, apc, _HP), jnp.int32),
            pltpu.SemaphoreType.DMA,
            pltpu.SemaphoreType.DMA,
            pltpu.SemaphoreType.DMA,
            pltpu.SemaphoreType.DMA,
        ],
        interpret=interpret,
    )
    def k(tbl_hbm, idx_hbm, out_hbm, idx_v, rows_v, acc_v,
          gsem0, gsem1, wsem0, wsem1):
        wid = lax.axis_index("s") * _NC + lax.axis_index("c")
        start, n_ch, max_ch = _split_chunks(wid, n_chunks)
        gsem = (gsem0, gsem1)
        wsem = (wsem0, wsem1)
        _stage_idx(idx_hbm, idx_v, start, n_ch, b, cpb)

        def issue_gather(kc, slot):
            @pl.when(kc < n_ch)
            def _():
                for g in range(cpb // _CHUNK):
                    off = pl.multiple_of(kc * cpb + g * _CHUNK, 128)
                    pltpu.async_copy(
                        tbl_hbm.at[idx_v.at[pl.ds(off, _CHUNK)]],
                        rows_v.at[slot, pl.ds(g * _CHUNK, _CHUNK)],
                        gsem[slot])

        def drain_gather(slot):
            for g in range(cpb // _CHUNK):
                pltpu.make_async_copy(
                    tbl_hbm.at[pl.ds(0, _CHUNK)],
                    rows_v.at[slot, pl.ds(g * _CHUNK, _CHUNK)],
                    gsem[slot]).wait()

        def drain_write(slot):
            pltpu.make_async_copy(acc_v.at[slot],
                                  out_hbm.at[pl.ds(0, apc)],
                                  wsem[slot]).wait()

        issue_gather(0, 0)

        def body(i, carry):
            for s in (0, 1):
                kc = 2 * i + s

                @pl.when(kc < n_ch)
                def _():
                    issue_gather(kc + 1, (s + 1) % 2)
                    drain_gather(s)

                    @pl.when(kc >= 2)
                    def _():
                        drain_write(s)

                    def rbody(r, rc):
                        for j in range(_HP // _LANES):
                            sl = pl.ds(j * _LANES, _LANES)
                            parts = [_sc_unpack(rows_v[s, r * n_nb + t, sl])
                                     for t in range(n_nb)]
                            acc_v[s, r, sl] = _sc_pack(
                                _tree_sum([p[0] for p in parts]),
                                _tree_sum([p[1] for p in parts]))
                        return rc

                    lax.fori_loop(0, apc, rbody, 0)
                    orow = pl.multiple_of((start + kc) * apc, 8)
                    pltpu.async_copy(acc_v.at[s],
                                     out_hbm.at[pl.ds(orow, apc)], wsem[s])

            return carry

        lax.fori_loop(0, (max_ch + 1) // 2, body, 0)
        drain_write(0)
        drain_write(1)

    return k(table64, idxg_flat)


def _pairmsg(inp64, p64, a64, b2a_po, brev_po, interpret=False):
    n_rows = inp64.shape[0]
    n_chunks = n_rows // _CHUNK
    b = n_chunks // _NW

    @functools.partial(
        pl.kernel,
        out_type=jax.ShapeDtypeStruct((n_rows, _HP), jnp.int32),
        mesh=_sc_mesh(),
        compiler_params=_SC_PARAMS,
        scratch_types=[
            pltpu.VMEM(((b + 1) * _CHUNK,), jnp.int32),
            pltpu.VMEM(((b + 1) * _CHUNK,), jnp.int32),
            pltpu.VMEM((2, _CHUNK, _HP), jnp.int32),
            pltpu.VMEM((2, _CHUNK, _HP), jnp.int32),
            pltpu.VMEM((2, _CHUNK, _HP), jnp.int32),
            pltpu.SemaphoreType.DMA,
            pltpu.SemaphoreType.DMA,
            pltpu.SemaphoreType.DMA,
            pltpu.SemaphoreType.DMA,
        ],
        interpret=interpret,
    )
    def k(inp_hbm, p_hbm, a_hbm, b2a_hbm, brev_hbm, out_hbm,
          idxa_v, idxr_v, rowsa_v, rowsr_v, inp_v,
          gsem0, gsem1, wsem0, wsem1):
        wid = lax.axis_index("s") * _NC + lax.axis_index("c")
        start, n_ch, max_ch = _split_chunks(wid, n_chunks)
        gsem = (gsem0, gsem1)
        wsem = (wsem0, wsem1)
        _stage_idx(b2a_hbm, idxa_v, start, n_ch, b, _CHUNK)
        _stage_idx(brev_hbm, idxr_v, start, n_ch, b, _CHUNK)

        def issue_ar(kc, slot):
            @pl.when(kc < n_ch)
            def _():
                off = pl.multiple_of(kc * _CHUNK, 128)
                pltpu.async_copy(a_hbm.at[idxa_v.at[pl.ds(off, _CHUNK)]],
                                 rowsa_v.at[slot], gsem[slot])
                pltpu.async_copy(p_hbm.at[idxr_v.at[pl.ds(off, _CHUNK)]],
                                 rowsr_v.at[slot], gsem[slot])

        def issue_inp(kc, slot):
            @pl.when(kc < n_ch)
            def _():
                row = pl.multiple_of((start + kc) * _CHUNK, 128)
                pltpu.async_copy(inp_hbm.at[pl.ds(row, _CHUNK)],
                                 inp_v.at[slot], gsem[slot])

        def drain_gathers(slot):
            dummy = pl.ds(0, _CHUNK)
            pltpu.make_async_copy(p_hbm.at[dummy], rowsa_v.at[slot],
                                  gsem[slot]).wait()
            pltpu.make_async_copy(p_hbm.at[dummy], rowsr_v.at[slot],
                                  gsem[slot]).wait()
            pltpu.make_async_copy(p_hbm.at[dummy], inp_v.at[slot],
                                  gsem[slot]).wait()

        def drain_write(slot):
            pltpu.make_async_copy(inp_v.at[slot],
                                  out_hbm.at[pl.ds(0, _CHUNK)],
                                  wsem[slot]).wait()

        issue_ar(0, 0)
        issue_inp(0, 0)

        def body(i, carry):
            for s in (0, 1):
                kc = 2 * i + s
                s2 = (s + 1) % 2

                @pl.when(kc < n_ch)
                def _():
                    issue_ar(kc + 1, s2)
                    drain_gathers(s)

                    @pl.when(jnp.logical_and(kc >= 1, kc + 1 < n_ch))
                    def _():
                        drain_write(s2)

                    issue_inp(kc + 1, s2)

                    def rbody(r, rc):
                        for j in range(_HP // _LANES):
                            sl = pl.ds(j * _LANES, _LANES)
                            ilo, ihi = _sc_unpack(inp_v[s, r, sl])
                            alo, ahi = _sc_unpack(rowsa_v[s, r, sl])
                            rlo, rhi = _sc_unpack(rowsr_v[s, r, sl])
                            inp_v[s, r, sl] = _sc_pack(
                                jnp.maximum(ilo + alo - rlo, 0.0),
                                jnp.maximum(ihi + ahi - rhi, 0.0))
                        return rc

                    lax.fori_loop(0, _CHUNK, rbody, 0)
                    row = pl.multiple_of((start + kc) * _CHUNK, 128)
                    pltpu.async_copy(inp_v.at[s],
                                     out_hbm.at[pl.ds(row, _CHUNK)], wsem[s])

            return carry

        lax.fori_loop(0, (max_ch + 1) // 2, body, 0)
        drain_write(0)
        drain_write(1)

    return k(inp64, p64, a64, b2a_po, brev_po)


def _mm_in(f_bonds, w_i, w_h, interpret=False):
    n, kdim = f_bonds.shape
    n2 = n // 2
    br2 = 1280
    assert n2 % br2 == 0

    def half(x_ref, wi, wh):
        i = jnp.dot(x_ref[...], wi, preferred_element_type=jnp.float32)
        ip = _tc_pack(i)
        lo, hi = _tc_unpack(ip)
        p0 = (jnp.dot(jnp.maximum(lo, 0.0), wh[:_HP],
                      preferred_element_type=jnp.float32)
              + jnp.dot(jnp.maximum(hi, 0.0), wh[_HP:],
                        preferred_element_type=jnp.float32))
        return ip, _tc_pack(p0)

    def body(xa_ref, xb_ref, wi_ref, wh_ref, inp_ref, p0_ref):
        wi = wi_ref[...]
        wh = wh_ref[...]
        ipa, p0a = half(xa_ref, wi, wh)
        ipb, p0b = half(xb_ref, wi, wh)
        inp_ref[...] = jnp.concatenate([ipa, ipb], axis=1)
        p0_ref[...] = jnp.concatenate([p0a, p0b], axis=1)

    nblk = n2 // br2
    return pl.pallas_call(
        body,
        grid=(nblk,),
        in_specs=[
            pl.BlockSpec((br2, kdim), lambda i: (i, 0)),
            pl.BlockSpec((br2, kdim), lambda i, _n=nblk: (i + _n, 0)),
            pl.BlockSpec((kdim, _H), lambda i: (0, 0)),
            pl.BlockSpec((_H, _H), lambda i: (0, 0)),
        ],
        out_specs=[
            pl.BlockSpec((br2, _H), lambda i: (i, 0)),
            pl.BlockSpec((br2, _H), lambda i: (i, 0)),
        ],
        out_shape=[
            jax.ShapeDtypeStruct((n2, _H), jnp.int32),
            jax.ShapeDtypeStruct((n2, _H), jnp.int32),
        ],
        interpret=interpret,
    )(f_bonds, f_bonds, w_i, w_h)


def _mm(x_packed, w, interpret=False):
    n2, _ = x_packed.shape
    br2 = 1280
    assert n2 % br2 == 0

    def body(x_ref, w_ref, o_ref):
        xi = x_ref[...]
        wv = w_ref[...]
        outs = []
        for h0 in (0, _HP):
            lo, hi = _tc_unpack(xi[:, h0:h0 + _HP])
            o = (jnp.dot(lo, wv[:_HP], preferred_element_type=jnp.float32)
                 + jnp.dot(hi, wv[_HP:], preferred_element_type=jnp.float32))
            outs.append(_tc_pack(o))
        o_ref[...] = jnp.concatenate(outs, axis=1)

    return pl.pallas_call(
        body,
        grid=(n2 // br2,),
        in_specs=[
            pl.BlockSpec((br2, _H), lambda i: (i, 0)),
            pl.BlockSpec((_H, _H), lambda i: (0, 0)),
        ],
        out_specs=pl.BlockSpec((br2, _H), lambda i: (i, 0)),
        out_shape=jax.ShapeDtypeStruct((n2, _H), jnp.int32),
        interpret=interpret,
    )(x_packed, w)


def _readout(f_atoms, a2_packed, mol_ids_f32, w_o1, w_o2, b_o, n_mols,
             interpret=False):
    n_atoms = f_atoms.shape[0]
    na2 = n_atoms // 2
    br2 = 1000
    assert na2 % br2 == 0
    steps = na2 // br2

    def body(fa_a, fa_b, am_ref, mid_a, mid_b, wo1_ref, wo2_ref, bo_ref,
             out_ref, sum_v, cnt_v):
        step = pl.program_id(0)

        @pl.when(step == 0)
        def _():
            sum_v[...] = jnp.zeros_like(sum_v)
            cnt_v[...] = jnp.zeros_like(cnt_v)

        am = am_ref[...]
        wo1 = wo1_ref[...]
        wo2 = wo2_ref[...]
        bo = bo_ref[...]
        iota = lax.broadcasted_iota(
            jnp.int32, (1, n_mols), 1).astype(jnp.float32)
        ones = jnp.ones((br2, 1), jnp.float32)
        for h0, fa_ref, mid_ref in ((0, fa_a, mid_a), (_HP, fa_b, mid_b)):
            lo, hi = _tc_unpack(am[:, h0:h0 + _HP])
            h = jnp.maximum(
                jnp.dot(fa_ref[...], wo1,
                        preferred_element_type=jnp.float32)
                + jnp.dot(lo, wo2[:_HP], preferred_element_type=jnp.float32)
                + jnp.dot(hi, wo2[_HP:], preferred_element_type=jnp.float32)
                + bo, 0.0)
            oh = (mid_ref[...] == iota).astype(jnp.float32)
            sum_v[...] += lax.dot_general(
                oh, h, (((0,), (0,)), ((), ())),
                preferred_element_type=jnp.float32)
            cnt_v[...] += lax.dot_general(
                oh, ones, (((0,), (0,)), ((), ())),
                preferred_element_type=jnp.float32)

        @pl.when(step == steps - 1)
        def _():
            out_ref[...] = sum_v[...] / jnp.maximum(cnt_v[...], 1.0)

    return pl.pallas_call(
        body,
        grid=(steps,),
        in_specs=[
            pl.BlockSpec((br2, _H), lambda i: (i, 0)),
            pl.BlockSpec((br2, _H), lambda i, _s=steps: (i + _s, 0)),
            pl.BlockSpec((br2, _H), lambda i: (i, 0)),
            pl.BlockSpec((br2, 1), lambda i: (i, 0)),
            pl.BlockSpec((br2, 1), lambda i, _s=steps: (i + _s, 0)),
            pl.BlockSpec((_H, _H), lambda i: (0, 0)),
            pl.BlockSpec((_H, _H), lambda i: (0, 0)),
            pl.BlockSpec((1, _H), lambda i: (0, 0)),
        ],
        out_specs=pl.BlockSpec((n_mols, _H), lambda i: (0, 0)),
        out_shape=jax.ShapeDtypeStruct((n_mols, _H), jnp.float32),
        scratch_shapes=[
            pltpu.VMEM((n_mols, _H), jnp.float32),
            pltpu.VMEM((n_mols, 1), jnp.float32),
        ],
        interpret=interpret,
    )(f_atoms, f_atoms, a2_packed, mol_ids_f32, mol_ids_f32,
      w_o1, w_o2, b_o)


def _interleave(x, half):
    return jnp.stack([x[:half], x[half:]], axis=1).reshape(
        (2 * half,) + x.shape[1:])


def _encode(f_atoms, f_bonds, a2b, b2a, b2revb, mol_ids, w_i, w_h, w_o, b_o,
            n_mols=256):
    n_bonds = f_bonds.shape[0]
    n_atoms = f_atoms.shape[0]
    n2 = n_bonds // 2
    na2 = n_atoms // 2
    n_nb = a2b.shape[1]

    def g(e):
        e = e.astype(jnp.int32)
        return jnp.where(e < n2, 2 * e, 2 * (e - n2) + 1)

    a2b_g = g(a2b)
    a2b_flat = a2b_g.reshape(-1)
    a2b_po = _interleave(a2b_g, na2).reshape(-1)
    b2a_po = _interleave(b2a.astype(jnp.int32), n2)
    brev_po = _interleave(g(b2revb), n2)

    inp_p, p0_p = _mm_in(f_bonds, w_i, w_h)
    inp64 = inp_p.reshape(n_bonds, _HP)
    p0_64 = p0_p.reshape(n_bonds, _HP)

    a0 = _nbsum(p0_64, a2b_flat, n_nb)
    msg1_64 = _pairmsg(inp64, p0_64, a0, b2a_po, brev_po)
    p1_p = _mm(msg1_64.reshape(n2, _H), w_h)
    p1_64 = p1_p.reshape(n_bonds, _HP)
    a1 = _nbsum(p1_64, a2b_flat, n_nb)
    msg2_64 = _pairmsg(inp64, p1_64, a1, b2a_po, brev_po)
    a2 = _nbsum(msg2_64, a2b_po, n_nb)
    a2_p = a2.reshape(na2, _H)

    mid = mol_ids.astype(jnp.float32).reshape(-1, 1)
    w_o1 = w_o[:f_atoms.shape[1], :]
    w_o2 = w_o[f_atoms.shape[1]:, :]
    return _readout(f_atoms, a2_p, mid, w_o1, w_o2, b_o.reshape(1, -1),
                    n_mols)


def kernel(f_atoms, f_bonds, a2b, b2a, b2revb, atom_mol_ids,
           ano_f_atoms, ano_f_bonds, ano_a2b, ano_b2a, ano_b2revb,
           ano_atom_mol_ids, W_i, W_h, W_o, b_o):
    mol_vecs = _encode(f_atoms, f_bonds, a2b, b2a, b2revb, atom_mol_ids,
                       W_i, W_h, W_o, b_o)
    ano_mol_vecs = _encode(ano_f_atoms, ano_f_bonds, ano_a2b, ano_b2a,
                           ano_b2revb, ano_atom_mol_ids, W_i, W_h, W_o, b_o)
    return (mol_vecs, ano_mol_vecs)
